# baseline (speedup 1.0000x reference)
import jax
import jax.numpy as jnp
from jax.experimental import pallas as pl
from jax.experimental.pallas import tpu as pltpu

_NUM_EXPERTS = 64
_TOP_K = 2
_N_TOKENS = 16384
_N_EMBD = 2048
_BLOCK = 512
_GRID = _N_TOKENS // _BLOCK
_LANES = 128


def _router_kernel(x_ref, wt_ref, scores_ref, idx_ref, aux_ref, accs_ref, accc_ref):
    step = pl.program_id(0)

    @pl.when(step == 0)
    def _init():
        accs_ref[...] = jnp.zeros_like(accs_ref)
        accc_ref[...] = jnp.zeros_like(accc_ref)

    logits = jnp.dot(x_ref[...], wt_ref[...], preferred_element_type=jnp.float32)
    lt = logits.T

    m = jnp.max(lt, axis=0, keepdims=True)
    iota = jax.lax.broadcasted_iota(jnp.int32, lt.shape, 0)
    i1 = jnp.min(jnp.where(lt == m, iota, _NUM_EXPERTS), axis=0, keepdims=True)
    c1 = iota == i1
    masked = jnp.where(c1, -jnp.inf, lt)
    m2 = jnp.max(masked, axis=0, keepdims=True)
    i2 = jnp.min(jnp.where(masked == m2, iota, _NUM_EXPERTS), axis=0, keepdims=True)
    c2 = iota == i2

    e = jnp.exp(lt - m)
    denom = jnp.sum(e, axis=0, keepdims=True)
    r = 1.0 / denom
    s2 = jnp.exp(m2 - m) * r
    st = e * r
    cnt = jnp.where(c1, 1.0, 0.0) + jnp.where(c2, 1.0, 0.0)

    ssum = st[:, 0:_LANES]
    csum = cnt[:, 0:_LANES]
    for k in range(1, _BLOCK // _LANES):
        ssum = ssum + st[:, k * _LANES:(k + 1) * _LANES]
        csum = csum + cnt[:, k * _LANES:(k + 1) * _LANES]
    accs_ref[...] += ssum
    accc_ref[...] += csum

    scores_ref[0:1, :] = r
    scores_ref[1:2, :] = s2
    idx_ref[0:1, :] = i1
    idx_ref[1:2, :] = i2

    @pl.when(step == _GRID - 1)
    def _finish():
        cs = jnp.sum(accs_ref[...], axis=1)
        ct = jnp.sum(accc_ref[...], axis=1)
        aux = jnp.sum(cs * ct) * _NUM_EXPERTS / (jnp.sum(cs) * jnp.sum(ct))
        aux_ref[...] = aux.reshape(1, 1)


def kernel(x, W):
    wt = W.T
    scores_t, idx_t, aux = pl.pallas_call(
        _router_kernel,
        grid=(_GRID,),
        in_specs=[
            pl.BlockSpec((_BLOCK, _N_EMBD), lambda i: (i, 0)),
            pl.BlockSpec((_N_EMBD, _NUM_EXPERTS), lambda i: (0, 0)),
        ],
        out_specs=[
            pl.BlockSpec((_TOP_K, _BLOCK), lambda i: (0, i)),
            pl.BlockSpec((_TOP_K, _BLOCK), lambda i: (0, i)),
            pl.BlockSpec((1, 1), lambda i: (0, 0)),
        ],
        out_shape=[
            jax.ShapeDtypeStruct((_TOP_K, _N_TOKENS), jnp.float32),
            jax.ShapeDtypeStruct((_TOP_K, _N_TOKENS), jnp.int32),
            jax.ShapeDtypeStruct((1, 1), jnp.float32),
        ],
        scratch_shapes=[
            pltpu.VMEM((_NUM_EXPERTS, _LANES), jnp.float32),
            pltpu.VMEM((_NUM_EXPERTS, _LANES), jnp.float32),
        ],
    )(x, wt)
    return scores_t.T, idx_t.T, aux[0, 0]


# speedup vs baseline: 1.1886x; 1.1886x over previous
import jax
import jax.numpy as jnp
from jax.experimental import pallas as pl
from jax.experimental.pallas import tpu as pltpu

_NUM_EXPERTS = 64
_TOP_K = 2
_N_TOKENS = 16384
_N_EMBD = 2048
_BLOCK = 1024
_GRID = _N_TOKENS // _BLOCK
_LANES = 128


def _router_kernel(xa_ref, xb_ref, wt_ref, scores_ref, idx_ref, aux_ref,
                   accs_ref, accc_ref):
    step = pl.program_id(0)

    @pl.when(step == 0)
    def _init():
        accs_ref[...] = jnp.zeros_like(accs_ref)
        accc_ref[...] = jnp.zeros_like(accc_ref)

    half = _N_EMBD // 2
    logits = (
        jnp.dot(xa_ref[...], wt_ref[0:half, :], preferred_element_type=jnp.float32)
        + jnp.dot(xb_ref[...], wt_ref[half:_N_EMBD, :], preferred_element_type=jnp.float32)
    )
    lt = logits.T

    m = jnp.max(lt, axis=0, keepdims=True)
    iota = jax.lax.broadcasted_iota(jnp.int32, lt.shape, 0)
    i1 = jnp.min(jnp.where(lt == m, iota, _NUM_EXPERTS), axis=0, keepdims=True)
    c1 = iota == i1
    masked = jnp.where(c1, -jnp.inf, lt)
    m2 = jnp.max(masked, axis=0, keepdims=True)
    i2 = jnp.min(jnp.where(masked == m2, iota, _NUM_EXPERTS), axis=0, keepdims=True)
    c2 = iota == i2

    e = jnp.exp(lt - m)
    denom = jnp.sum(e, axis=0, keepdims=True)
    r = 1.0 / denom
    s2 = jnp.exp(m2 - m) * r
    st = e * r
    cnt = jnp.where(c1, 1.0, 0.0) + jnp.where(c2, 1.0, 0.0)

    ssum = st[:, 0:_LANES]
    csum = cnt[:, 0:_LANES]
    for k in range(1, _BLOCK // _LANES):
        ssum = ssum + st[:, k * _LANES:(k + 1) * _LANES]
        csum = csum + cnt[:, k * _LANES:(k + 1) * _LANES]
    accs_ref[...] += ssum
    accc_ref[...] += csum

    scores_ref[0:1, :] = r
    scores_ref[1:2, :] = s2
    idx_ref[0:1, :] = i1
    idx_ref[1:2, :] = i2

    @pl.when(step == _GRID - 1)
    def _finish():
        cs = jnp.sum(accs_ref[...], axis=1)
        ct = jnp.sum(accc_ref[...], axis=1)
        aux = jnp.sum(cs * ct) * _NUM_EXPERTS / (jnp.sum(cs) * jnp.sum(ct))
        aux_ref[...] = aux.reshape(1, 1)


def kernel(x, W):
    wt = W.T
    scores_t, idx_t, aux = pl.pallas_call(
        _router_kernel,
        grid=(_GRID,),
        in_specs=[
            pl.BlockSpec((_BLOCK, _N_EMBD // 2), lambda i: (i, 0)),
            pl.BlockSpec((_BLOCK, _N_EMBD // 2), lambda i: (i, 1)),
            pl.BlockSpec((_N_EMBD, _NUM_EXPERTS), lambda i: (0, 0)),
        ],
        out_specs=[
            pl.BlockSpec((_TOP_K, _BLOCK), lambda i: (0, i)),
            pl.BlockSpec((_TOP_K, _BLOCK), lambda i: (0, i)),
            pl.BlockSpec((1, 1), lambda i: (0, 0)),
        ],
        out_shape=[
            jax.ShapeDtypeStruct((_TOP_K, _N_TOKENS), jnp.float32),
            jax.ShapeDtypeStruct((_TOP_K, _N_TOKENS), jnp.int32),
            jax.ShapeDtypeStruct((1, 1), jnp.float32),
        ],
        scratch_shapes=[
            pltpu.VMEM((_NUM_EXPERTS, _LANES), jnp.float32),
            pltpu.VMEM((_NUM_EXPERTS, _LANES), jnp.float32),
        ],
    )(x, x, wt)
    return scores_t.T, idx_t.T, aux[0, 0]


# speedup vs baseline: 1.1959x; 1.0061x over previous
import jax
import jax.numpy as jnp
from jax.experimental import pallas as pl
from jax.experimental.pallas import tpu as pltpu

_NUM_EXPERTS = 64
_TOP_K = 2
_N_TOKENS = 16384
_N_EMBD = 2048
_BLOCK = 1024
_GRID = _N_TOKENS // _BLOCK
_LANES = 128


def _router_kernel(x_ref, wt_ref, scores_ref, idx_ref, aux_ref,
                   accs_ref, accc_ref):
    step = pl.program_id(0)

    @pl.when(step == 0)
    def _init():
        accs_ref[...] = jnp.zeros_like(accs_ref)
        accc_ref[...] = jnp.zeros_like(accc_ref)

    logits = jnp.dot(x_ref[...], wt_ref[...], preferred_element_type=jnp.float32)
    lt = logits.T

    m = jnp.max(lt, axis=0, keepdims=True)
    iota = jax.lax.broadcasted_iota(jnp.int32, lt.shape, 0)
    i1 = jnp.min(jnp.where(lt == m, iota, _NUM_EXPERTS), axis=0, keepdims=True)
    c1 = iota == i1
    masked = jnp.where(c1, -jnp.inf, lt)
    m2 = jnp.max(masked, axis=0, keepdims=True)
    i2 = jnp.min(jnp.where(masked == m2, iota, _NUM_EXPERTS), axis=0, keepdims=True)
    c2 = iota == i2

    e = jnp.exp(lt - m)
    denom = jnp.sum(e, axis=0, keepdims=True)
    r = 1.0 / denom
    s2 = jnp.exp(m2 - m) * r
    st = e * r
    cnt = jnp.where(c1 | c2, 1.0, 0.0)

    ssum = st[:, 0:_LANES]
    csum = cnt[:, 0:_LANES]
    for k in range(1, _BLOCK // _LANES):
        ssum = ssum + st[:, k * _LANES:(k + 1) * _LANES]
        csum = csum + cnt[:, k * _LANES:(k + 1) * _LANES]
    accs_ref[...] += ssum
    accc_ref[...] += csum

    scores_ref[0:1, :] = r
    scores_ref[1:2, :] = s2
    idx_ref[0:1, :] = i1
    idx_ref[1:2, :] = i2

    @pl.when(step == _GRID - 1)
    def _finish():
        cs = jnp.sum(accs_ref[...], axis=1)
        ct = jnp.sum(accc_ref[...], axis=1)
        aux = jnp.sum(cs * ct) * _NUM_EXPERTS / (jnp.sum(cs) * jnp.sum(ct))
        aux_ref[...] = aux.reshape(1, 1)


def kernel(x, W):
    wt = W.T
    scores_t, idx_t, aux = pl.pallas_call(
        _router_kernel,
        grid=(_GRID,),
        in_specs=[
            pl.BlockSpec((_BLOCK, _N_EMBD), lambda i: (i, 0)),
            pl.BlockSpec((_N_EMBD, _NUM_EXPERTS), lambda i: (0, 0)),
        ],
        out_specs=[
            pl.BlockSpec((_TOP_K, _BLOCK), lambda i: (0, i)),
            pl.BlockSpec((_TOP_K, _BLOCK), lambda i: (0, i)),
            pl.BlockSpec((1, 1), lambda i: (0, 0)),
        ],
        out_shape=[
            jax.ShapeDtypeStruct((_TOP_K, _N_TOKENS), jnp.float32),
            jax.ShapeDtypeStruct((_TOP_K, _N_TOKENS), jnp.int32),
            jax.ShapeDtypeStruct((1, 1), jnp.float32),
        ],
        scratch_shapes=[
            pltpu.VMEM((_NUM_EXPERTS, _LANES), jnp.float32),
            pltpu.VMEM((_NUM_EXPERTS, _LANES), jnp.float32),
        ],
    )(x, wt)
    return scores_t.T, idx_t.T, aux[0, 0]
